# Initial kernel scaffold; baseline (speedup 1.0000x reference)
import jax
import jax.numpy as jnp
from jax.experimental import pallas as pl


def kernel(x, means, log_proportions, labels, neighborhood_ids, closest_neighbors, unit_search_neighbors, explore_targs, neighborhood_explore_units):
    raise NotImplementedError("write your pallas kernel here")


# speedup vs baseline: 10.5902x; 10.5902x over previous
import functools

import jax
import jax.numpy as jnp
from jax import lax
from jax.experimental import pallas as pl
from jax.experimental.pallas import tpu as pltpu

C = 3
NS = 2
T = C + C * NS + 1


def _pick_bn(n):
    for bn in (1000, 800, 500, 400, 320, 250, 200, 160, 100, 80, 50, 40, 25, 20, 10, 8, 5, 4, 2, 1):
        if n % bn == 0 and bn % 8 == 0:
            return bn
    return n


def _body(x_ref, mt_ref, lp_ref, cn_ref, usn_ref, neu_ref, lab_ref, nb_ref,
          targ_ref, q_ref, top_ref, counts_ref, acc_ref):
    i = pl.program_id(0)
    bn = x_ref.shape[0]
    u = mt_ref.shape[1]
    xp = neu_ref.shape[1]
    nbb = neu_ref.shape[0]

    mt = mt_ref[:]
    xm = jnp.dot(x_ref[:], mt, preferred_element_type=jnp.float32)
    mu2 = jnp.sum(mt * mt, axis=0, keepdims=True)
    score = xm + (lp_ref[:] - 0.5 * mu2)

    iota_u = lax.broadcasted_iota(jnp.int32, (bn, u), 1).astype(jnp.float32)

    labf = lab_ref[:].astype(jnp.float32)
    oh_lab = jnp.where(iota_u == labf, 1.0, 0.0)
    top3 = jnp.dot(oh_lab, cn_ref[:], preferred_element_type=jnp.float32,
                   precision=lax.Precision.HIGHEST)

    cand_cols = [top3[:, c:c + 1] for c in range(C)]

    search_cols = []
    for c in range(C):
        ohc = jnp.where(iota_u == cand_cols[c], 1.0, 0.0)
        sc = jnp.dot(ohc, usn_ref[:], preferred_element_type=jnp.float32,
                     precision=lax.Precision.HIGHEST)
        for s in range(NS):
            search_cols.append(sc[:, s:s + 1])

    iota_nb = lax.broadcasted_iota(jnp.int32, (bn, nbb), 1).astype(jnp.float32)
    oh_nb = jnp.where(iota_nb == nb_ref[:].astype(jnp.float32), 1.0, 0.0)
    pool = jnp.dot(oh_nb, neu_ref[:], preferred_element_type=jnp.float32,
                   precision=lax.Precision.HIGHEST)
    iota_xp = lax.broadcasted_iota(jnp.int32, (bn, xp), 1).astype(jnp.float32)
    targf = targ_ref[:].astype(jnp.float32)
    explore = jnp.sum(jnp.where(iota_xp == targf, pool, 0.0), axis=1,
                      keepdims=True)

    cand_cols = cand_cols + search_cols + [explore]

    g_cols = []
    oh_sum = None
    for t in range(T):
        oh = jnp.where(iota_u == cand_cols[t], 1.0, 0.0)
        oh_sum = oh if oh_sum is None else oh_sum + oh
        g_cols.append(jnp.sum(oh * score, axis=1, keepdims=True))

    gvals = jnp.concatenate(g_cols, axis=1)
    cand = jnp.concatenate(cand_cols, axis=1)
    iota_t = lax.broadcasted_iota(jnp.int32, (bn, T), 1).astype(jnp.float32)

    cur = gvals
    vals, ids = [], []
    for _ in range(C):
        m = jnp.max(cur, axis=1, keepdims=True)
        ti = jnp.min(jnp.where(cur == m, iota_t, float(T)), axis=1,
                     keepdims=True)
        sel = iota_t == ti
        vals.append(m)
        ids.append(jnp.sum(jnp.where(sel, cand, 0.0), axis=1, keepdims=True))
        cur = jnp.where(sel, -jnp.inf, cur)

    vcat = jnp.concatenate(vals, axis=1)
    e = jnp.exp(vcat - vals[0])
    q_ref[:] = e / jnp.sum(e, axis=1, keepdims=True)
    top_ref[:] = jnp.concatenate(ids, axis=1).astype(jnp.int32)

    cblk = lax.dot_general(oh_sum, oh_nb, (((0,), (0,)), ((), ())),
                           preferred_element_type=jnp.float32)

    @pl.when(i == 0)
    def _():
        acc_ref[:] = cblk

    @pl.when(i > 0)
    def _():
        acc_ref[:] = acc_ref[:] + cblk

    @pl.when(i == pl.num_programs(0) - 1)
    def _():
        counts_ref[:] = acc_ref[:].astype(jnp.int32)


@functools.partial(jax.jit, static_argnames=("interpret",))
def _run(x, means, log_proportions, labels, neighborhood_ids,
         closest_neighbors, unit_search_neighbors, explore_targs,
         neighborhood_explore_units, interpret=False):
    n, d = x.shape
    u = means.shape[0]
    nbb, xp = neighborhood_explore_units.shape
    bn = _pick_bn(n)
    grid = (n // bn,)

    mt = means.T
    lp = log_proportions.reshape(1, u)
    cn = closest_neighbors.astype(jnp.float32)
    usn = unit_search_neighbors.astype(jnp.float32)
    neu = neighborhood_explore_units.astype(jnp.float32)
    lab = labels.reshape(n, 1)
    nb = neighborhood_ids.reshape(n, 1)
    targ = explore_targs.reshape(n, 1)

    q, top, counts = pl.pallas_call(
        _body,
        grid=grid,
        in_specs=[
            pl.BlockSpec((bn, d), lambda i: (i, 0)),
            pl.BlockSpec((d, u), lambda i: (0, 0)),
            pl.BlockSpec((1, u), lambda i: (0, 0)),
            pl.BlockSpec((u, C), lambda i: (0, 0)),
            pl.BlockSpec((u, NS), lambda i: (0, 0)),
            pl.BlockSpec((nbb, xp), lambda i: (0, 0)),
            pl.BlockSpec((bn, 1), lambda i: (i, 0)),
            pl.BlockSpec((bn, 1), lambda i: (i, 0)),
            pl.BlockSpec((bn, 1), lambda i: (i, 0)),
        ],
        out_specs=[
            pl.BlockSpec((bn, C), lambda i: (i, 0)),
            pl.BlockSpec((bn, C), lambda i: (i, 0)),
            pl.BlockSpec((u, nbb), lambda i: (0, 0)),
        ],
        out_shape=[
            jax.ShapeDtypeStruct((n, C), jnp.float32),
            jax.ShapeDtypeStruct((n, C), jnp.int32),
            jax.ShapeDtypeStruct((u, nbb), jnp.int32),
        ],
        scratch_shapes=[pltpu.VMEM((u, nbb), jnp.float32)],
        compiler_params=pltpu.CompilerParams(
            dimension_semantics=("arbitrary",)),
        interpret=interpret,
    )(x, mt, lp, cn, usn, neu, lab, nb, targ)
    return q, top, counts


def kernel(x, means, log_proportions, labels, neighborhood_ids,
           closest_neighbors, unit_search_neighbors, explore_targs,
           neighborhood_explore_units):
    return _run(x, means, log_proportions, labels, neighborhood_ids,
                closest_neighbors, unit_search_neighbors, explore_targs,
                neighborhood_explore_units)


# speedup vs baseline: 18.0964x; 1.7088x over previous
import functools

import jax
import jax.numpy as jnp
from jax import lax
from jax.experimental import pallas as pl
from jax.experimental.pallas import tpu as pltpu

C = 3
NS = 2
T = C + C * NS + 1


def _pick_bn(n):
    for bn in (1000, 800, 500, 400, 320, 250, 200, 160, 100, 80, 50, 40, 25, 20, 10, 8, 5, 4, 2, 1):
        if n % bn == 0 and bn % 8 == 0:
            return bn
    return n


def _body(x_ref, mt_ref, lp_ref, cn_ref, usn_ref, neu_ref, lab_ref, nb_ref,
          targ_ref, q_ref, top_ref, counts_ref, acc_ref):
    i = pl.program_id(0)
    bn = x_ref.shape[0]
    u = mt_ref.shape[1]
    xp = neu_ref.shape[1] // 2
    nbb = neu_ref.shape[0]

    mt = mt_ref[:]
    xm = jnp.dot(x_ref[:], mt, preferred_element_type=jnp.float32)
    mu2 = jnp.sum(mt * mt, axis=0, keepdims=True)
    score = xm + (lp_ref[:] - 0.5 * mu2)

    bf16 = jnp.bfloat16
    iota_u = lax.broadcasted_iota(jnp.int32, (bn, u), 1).astype(jnp.float32)

    def unsplit(hl, w):
        return 16.0 * hl[:, :w] + hl[:, w:2 * w]

    labf = lab_ref[:].astype(jnp.float32)
    oh_lab = jnp.where(iota_u == labf, 1.0, 0.0)
    top3 = unsplit(jnp.dot(oh_lab.astype(bf16), cn_ref[:],
                           preferred_element_type=jnp.float32), C)

    cand_cols = [top3[:, c:c + 1] for c in range(C)]
    ohs = [jnp.where(iota_u == cand_cols[c], 1.0, 0.0) for c in range(C)]

    for c in range(C):
        sc = unsplit(jnp.dot(ohs[c].astype(bf16), usn_ref[:],
                             preferred_element_type=jnp.float32), NS)
        for s in range(NS):
            col = sc[:, s:s + 1]
            cand_cols.append(col)
            ohs.append(jnp.where(iota_u == col, 1.0, 0.0))

    iota_nb = lax.broadcasted_iota(jnp.int32, (bn, nbb), 1).astype(jnp.float32)
    oh_nb = jnp.where(iota_nb == nb_ref[:].astype(jnp.float32), 1.0, 0.0)
    pool = unsplit(jnp.dot(oh_nb.astype(bf16), neu_ref[:],
                           preferred_element_type=jnp.float32), xp)
    iota_xp = lax.broadcasted_iota(jnp.int32, (bn, xp), 1).astype(jnp.float32)
    targf = targ_ref[:].astype(jnp.float32)
    explore = jnp.sum(jnp.where(iota_xp == targf, pool, 0.0), axis=1,
                      keepdims=True)
    cand_cols.append(explore)
    ohs.append(jnp.where(iota_u == explore, 1.0, 0.0))

    g_cols = []
    oh_sum = None
    for t in range(T):
        oh_sum = ohs[t] if oh_sum is None else oh_sum + ohs[t]
        g_cols.append(jnp.sum(ohs[t] * score, axis=1, keepdims=True))

    gvals = jnp.concatenate(g_cols, axis=1)
    cand = jnp.concatenate(cand_cols, axis=1)
    iota_t = lax.broadcasted_iota(jnp.int32, (bn, T), 1).astype(jnp.float32)

    cur = gvals
    vals, ids = [], []
    for _ in range(C):
        m = jnp.max(cur, axis=1, keepdims=True)
        ti = jnp.min(jnp.where(cur == m, iota_t, float(T)), axis=1,
                     keepdims=True)
        sel = iota_t == ti
        vals.append(m)
        ids.append(jnp.sum(jnp.where(sel, cand, 0.0), axis=1, keepdims=True))
        cur = jnp.where(sel, -jnp.inf, cur)

    vcat = jnp.concatenate(vals, axis=1)
    e = jnp.exp(vcat - vals[0])
    q_ref[:] = e / jnp.sum(e, axis=1, keepdims=True)
    top_ref[:] = jnp.concatenate(ids, axis=1).astype(jnp.int32)

    cblk = lax.dot_general(oh_sum.astype(bf16), oh_nb.astype(bf16),
                           (((0,), (0,)), ((), ())),
                           preferred_element_type=jnp.float32)

    @pl.when(i == 0)
    def _():
        acc_ref[:] = cblk

    @pl.when(i > 0)
    def _():
        acc_ref[:] = acc_ref[:] + cblk

    @pl.when(i == pl.num_programs(0) - 1)
    def _():
        counts_ref[:] = acc_ref[:].astype(jnp.int32)


@functools.partial(jax.jit, static_argnames=("interpret",))
def _run(x, means, log_proportions, labels, neighborhood_ids,
         closest_neighbors, unit_search_neighbors, explore_targs,
         neighborhood_explore_units, interpret=False):
    n, d = x.shape
    u = means.shape[0]
    nbb, xp = neighborhood_explore_units.shape
    bn = _pick_bn(n)
    grid = (n // bn,)

    mt = means.T
    lp = log_proportions.reshape(1, u)

    def split16(t):
        return jnp.concatenate([t // 16, t % 16], axis=1).astype(jnp.bfloat16)

    cn = split16(closest_neighbors)
    usn = split16(unit_search_neighbors)
    neu = split16(neighborhood_explore_units)
    lab = labels.reshape(n, 1)
    nb = neighborhood_ids.reshape(n, 1)
    targ = explore_targs.reshape(n, 1)

    q, top, counts = pl.pallas_call(
        _body,
        grid=grid,
        in_specs=[
            pl.BlockSpec((bn, d), lambda i: (i, 0)),
            pl.BlockSpec((d, u), lambda i: (0, 0)),
            pl.BlockSpec((1, u), lambda i: (0, 0)),
            pl.BlockSpec((u, 2 * C), lambda i: (0, 0)),
            pl.BlockSpec((u, 2 * NS), lambda i: (0, 0)),
            pl.BlockSpec((nbb, 2 * xp), lambda i: (0, 0)),
            pl.BlockSpec((bn, 1), lambda i: (i, 0)),
            pl.BlockSpec((bn, 1), lambda i: (i, 0)),
            pl.BlockSpec((bn, 1), lambda i: (i, 0)),
        ],
        out_specs=[
            pl.BlockSpec((bn, C), lambda i: (i, 0)),
            pl.BlockSpec((bn, C), lambda i: (i, 0)),
            pl.BlockSpec((u, nbb), lambda i: (0, 0)),
        ],
        out_shape=[
            jax.ShapeDtypeStruct((n, C), jnp.float32),
            jax.ShapeDtypeStruct((n, C), jnp.int32),
            jax.ShapeDtypeStruct((u, nbb), jnp.int32),
        ],
        scratch_shapes=[pltpu.VMEM((u, nbb), jnp.float32)],
        compiler_params=pltpu.CompilerParams(
            dimension_semantics=("arbitrary",)),
        interpret=interpret,
    )(x, mt, lp, cn, usn, neu, lab, nb, targ)
    return q, top, counts


def kernel(x, means, log_proportions, labels, neighborhood_ids,
           closest_neighbors, unit_search_neighbors, explore_targs,
           neighborhood_explore_units):
    return _run(x, means, log_proportions, labels, neighborhood_ids,
                closest_neighbors, unit_search_neighbors, explore_targs,
                neighborhood_explore_units)


# speedup vs baseline: 40.9691x; 2.2639x over previous
import functools

import jax
import jax.numpy as jnp
from jax import lax
from jax.experimental import pallas as pl
from jax.experimental.pallas import tpu as pltpu
from jax.experimental.pallas import tpu_sc as plsc

C = 3
NS = 2
T = C + C * NS + 1

NW = 32
CHUNK = 3200
HALF = 1600
NGRP = HALF // 16
NDMA = HALF * T // 128
PIPE = 8


def _score_body(x_ref, mt_ref, lp_ref, out_ref):
    mt = mt_ref[:]
    mu2 = jnp.sum(mt * mt, axis=0, keepdims=True)
    out_ref[:] = jnp.dot(x_ref[:], mt, preferred_element_type=jnp.float32) \
        + (lp_ref[:] - 0.5 * mu2)


def _make_score(n, d, u, bn):
    return pl.pallas_call(
        _score_body,
        grid=(n // bn,),
        in_specs=[
            pl.BlockSpec((bn, d), lambda i: (i, 0)),
            pl.BlockSpec((d, u), lambda i: (0, 0)),
            pl.BlockSpec((1, u), lambda i: (0, 0)),
        ],
        out_specs=pl.BlockSpec((bn, u), lambda i: (i, 0)),
        out_shape=jax.ShapeDtypeStruct((n, u), jnp.float32),
        compiler_params=pltpu.CompilerParams(
            dimension_semantics=("parallel",)),
    )


def _sc_body(n, u, nbb, xp,
             lab_h, nb_h, targ_h, cn_h, usn_h, neu_h, score_h,
             q0_h, q1_h, q2_h, i0_h, i1_h, i2_h, cnt_h,
             lab_v, nb_v, targ_v, cn_v, usn_v, neu_v,
             idx_v, gath_v, q0_v, q1_v, q2_v, i0_v, i1_v, i2_v,
             hist_v, rowidx_v, shared_v, sem):
    sid = lax.axis_index("s")
    cid = lax.axis_index("c")
    wid = sid * 2 + cid
    base = wid * CHUNK

    pltpu.sync_copy(lab_h.at[pl.ds(base, CHUNK)], lab_v)
    pltpu.sync_copy(nb_h.at[pl.ds(base, CHUNK)], nb_v)
    pltpu.sync_copy(targ_h.at[pl.ds(base, CHUNK)], targ_v)
    pltpu.sync_copy(cn_h, cn_v)
    pltpu.sync_copy(usn_h, usn_v)
    pltpu.sync_copy(neu_h, neu_v)

    lanes = lax.broadcasted_iota(jnp.int32, (16,), 0)
    zeros16 = jnp.zeros((16,), jnp.int32)
    ones16 = jnp.ones((16,), jnp.int32)

    hrows = (u * nbb) // 128

    def _zh(j, _):
        r = j // 8
        col = (j % 8) * 16
        hist_v[r, pl.ds(col, 16)] = zeros16
        return 0
    lax.fori_loop(0, hrows * 8, _zh, 0)

    @pl.when(sid == 0)
    def _():
        pltpu.sync_copy(hist_v, shared_v)
    plsc.subcore_barrier()

    for h in range(CHUNK // HALF):
        hbase = h * HALF

        def _p1(g, _):
            off = hbase + g * 16
            lab = lab_v[pl.ds(off, 16)]
            nb = nb_v[pl.ds(off, 16)]
            tg = targ_v[pl.ds(off, 16)]
            spike = base + off + lanes
            sp = jnp.minimum(spike, n - 1)
            valid = spike < n
            cands = []
            for c in range(C):
                cands.append(plsc.load_gather(cn_v, [lab * C + c]))
            for c in range(C):
                for s in range(NS):
                    cands.append(plsc.load_gather(usn_v, [cands[c] * NS + s]))
            cands.append(plsc.load_gather(neu_v, [nb * xp + tg]))
            for t in range(T):
                bin_ = cands[t] * nbb + nb
                plsc.addupdate_scatter(
                    hist_v, [jnp.right_shift(bin_, 7),
                             jnp.bitwise_and(bin_, 127)],
                    ones16, mask=valid)
                idx_v[pl.ds(t * HALF + g * 16, 16)] = sp * u + cands[t]
            return 0
        lax.fori_loop(0, NGRP, _p1, 0)

        def _dma(j):
            src = score_h.at[idx_v.at[pl.ds(j * 128, 128)]]
            dst = gath_v.at[pl.ds(j * 128, 128)]
            return pltpu.make_async_copy(src, dst, sem)

        def _pg(j, _):
            _dma(j).start()

            @pl.when(j >= PIPE)
            def _():
                _dma(j - PIPE).wait()
            return 0
        lax.fori_loop(0, NDMA, _pg, 0)
        for k in range(PIPE):
            _dma(NDMA - PIPE + k).wait()

        def _p2(g, _):
            off16 = g * 16
            goff = hbase + off16
            gv = [gath_v[pl.ds(t * HALF + off16, 16)] for t in range(T)]
            ci = [jnp.bitwise_and(idx_v[pl.ds(t * HALF + off16, 16)], u - 1)
                  for t in range(T)]
            cur = list(gv)
            neg = jnp.float32(-3e38)
            vals, ids = [], []
            for _k in range(C):
                m = cur[0]
                for t in range(1, T):
                    m = jnp.maximum(m, cur[t])
                ti = jnp.full((16,), T, jnp.int32)
                for t in range(T - 1, -1, -1):
                    ti = jnp.where(cur[t] == m, t, ti)
                idk = zeros16
                for t in range(T):
                    idk = jnp.where(ti == t, ci[t], idk)
                    cur[t] = jnp.where(ti == t, neg, cur[t])
                vals.append(m)
                ids.append(idk)
            e2 = jnp.exp(vals[1] - vals[0])
            e3 = jnp.exp(vals[2] - vals[0])
            s = 1.0 + e2 + e3
            q0_v[pl.ds(goff, 16)] = 1.0 / s
            q1_v[pl.ds(goff, 16)] = e2 / s
            q2_v[pl.ds(goff, 16)] = e3 / s
            i0_v[pl.ds(goff, 16)] = ids[0]
            i1_v[pl.ds(goff, 16)] = ids[1]
            i2_v[pl.ds(goff, 16)] = ids[2]
            return 0
        lax.fori_loop(0, NGRP, _p2, 0)

    pltpu.sync_copy(q0_v, q0_h.at[pl.ds(base, CHUNK)])
    pltpu.sync_copy(q1_v, q1_h.at[pl.ds(base, CHUNK)])
    pltpu.sync_copy(q2_v, q2_h.at[pl.ds(base, CHUNK)])
    pltpu.sync_copy(i0_v, i0_h.at[pl.ds(base, CHUNK)])
    pltpu.sync_copy(i1_v, i1_h.at[pl.ds(base, CHUNK)])
    pltpu.sync_copy(i2_v, i2_h.at[pl.ds(base, CHUNK)])

    for j in range(hrows // 128):
        for k in range(8):
            rowidx_v[j, pl.ds(k * 16, 16)] = j * 128 + k * 16 + lanes
    for j in range(hrows // 128):
        pltpu.sync_copy(hist_v.at[pl.ds(j * 128, 128)],
                        shared_v.at[rowidx_v.at[j]], add=True)
    plsc.subcore_barrier()

    @pl.when(sid == 0)
    def _():
        pltpu.sync_copy(shared_v, cnt_h.at[cid])


@jax.jit
def _run(x, means, log_proportions, labels, neighborhood_ids,
         closest_neighbors, unit_search_neighbors, explore_targs,
         neighborhood_explore_units):
    n, d = x.shape
    u = means.shape[0]
    nbb, xp = neighborhood_explore_units.shape
    npad = NW * CHUNK
    hrows = (u * nbb) // 128

    mt = means.T
    lp = log_proportions.reshape(1, u)
    score = _make_score(n, d, u, 2000)(x, mt, lp)

    pad = (0, npad - n)
    lab_p = jnp.pad(labels, pad)
    nb_p = jnp.pad(neighborhood_ids, pad)
    targ_p = jnp.pad(explore_targs.reshape(-1), pad)

    sc = functools.partial(
        pl.kernel,
        mesh=plsc.VectorSubcoreMesh(core_axis_name="c", subcore_axis_name="s"),
        compiler_params=pltpu.CompilerParams(needs_layout_passes=False),
        out_type=[
            jax.ShapeDtypeStruct((npad,), jnp.float32),
            jax.ShapeDtypeStruct((npad,), jnp.float32),
            jax.ShapeDtypeStruct((npad,), jnp.float32),
            jax.ShapeDtypeStruct((npad,), jnp.int32),
            jax.ShapeDtypeStruct((npad,), jnp.int32),
            jax.ShapeDtypeStruct((npad,), jnp.int32),
            jax.ShapeDtypeStruct((2, hrows, 128), jnp.int32),
        ],
        scratch_types=[
            pltpu.VMEM((CHUNK,), jnp.int32),
            pltpu.VMEM((CHUNK,), jnp.int32),
            pltpu.VMEM((CHUNK,), jnp.int32),
            pltpu.VMEM((u * C,), jnp.int32),
            pltpu.VMEM((u * NS,), jnp.int32),
            pltpu.VMEM((nbb * xp,), jnp.int32),
            pltpu.VMEM((HALF * T,), jnp.int32),
            pltpu.VMEM((HALF * T,), jnp.float32),
            pltpu.VMEM((CHUNK,), jnp.float32),
            pltpu.VMEM((CHUNK,), jnp.float32),
            pltpu.VMEM((CHUNK,), jnp.float32),
            pltpu.VMEM((CHUNK,), jnp.int32),
            pltpu.VMEM((CHUNK,), jnp.int32),
            pltpu.VMEM((CHUNK,), jnp.int32),
            pltpu.VMEM((hrows, 128), jnp.int32),
            pltpu.VMEM((hrows // 128, 128), jnp.int32),
            pltpu.VMEM_SHARED((hrows, 128), jnp.int32),
            pltpu.SemaphoreType.DMA,
        ],
    )(functools.partial(_sc_body, n, u, nbb, xp))

    q0, q1, q2, i0, i1, i2, cnt = sc(
        lab_p, nb_p, targ_p,
        closest_neighbors.reshape(-1), unit_search_neighbors.reshape(-1),
        neighborhood_explore_units.reshape(-1), score.reshape(-1))

    q = jnp.stack([q0[:n], q1[:n], q2[:n]], axis=1)
    top = jnp.stack([i0[:n], i1[:n], i2[:n]], axis=1)
    counts = cnt.sum(axis=0).reshape(u, nbb)
    return q, top, counts


def kernel(x, means, log_proportions, labels, neighborhood_ids,
           closest_neighbors, unit_search_neighbors, explore_targs,
           neighborhood_explore_units):
    return _run(x, means, log_proportions, labels, neighborhood_ids,
                closest_neighbors, unit_search_neighbors, explore_targs,
                neighborhood_explore_units)


# speedup vs baseline: 58.5931x; 1.4302x over previous
import functools

import jax
import jax.numpy as jnp
from jax import lax
from jax.experimental import pallas as pl
from jax.experimental.pallas import tpu as pltpu
from jax.experimental.pallas import tpu_sc as plsc

C = 3
NS = 2
T = C + C * NS + 1

NW = 32
CHUNK = 3200
HALF = 1600
NGRP = HALF // 16
NDMA = HALF * T // 128
PIPE = 8


def _score_body(x_ref, mt_ref, lp_ref, out_ref):
    mt = mt_ref[:]
    mu2 = jnp.sum(mt * mt, axis=0, keepdims=True)
    score = jnp.dot(x_ref[:], mt, preferred_element_type=jnp.float32) \
        + (lp_ref[:] - 0.5 * mu2)
    for q in range(out_ref.shape[0]):
        out_ref[q, :, :] = score[:, q * 128:(q + 1) * 128]


def _make_score(n, d, u, bn):
    nq = u // 128
    return pl.pallas_call(
        _score_body,
        grid=(n // bn,),
        in_specs=[
            pl.BlockSpec((bn, d), lambda i: (i, 0)),
            pl.BlockSpec((d, u), lambda i: (0, 0)),
            pl.BlockSpec((1, u), lambda i: (0, 0)),
        ],
        out_specs=pl.BlockSpec((nq, bn, 128), lambda i: (0, i, 0)),
        out_shape=jax.ShapeDtypeStruct((nq, n, 128), jnp.float32),
        compiler_params=pltpu.CompilerParams(
            dimension_semantics=("parallel",)),
    )


def _sc_body(n, u, nbb, xp,
             lab_h, nb_h, targ_h, cn_h, usn_h, neu_h, score_h,
             q0_h, q1_h, q2_h, i0_h, i1_h, i2_h, cnt_h,
             lab_v, nb_v, targ_v, cn_v, usn_v, neu_v,
             idx_v, cand_v, gath_v, q0_v, q1_v, q2_v, i0_v, i1_v, i2_v,
             hist_v, rowidx_v, shared_v, sem):
    sid = lax.axis_index("s")
    cid = lax.axis_index("c")
    wid = sid * 2 + cid
    base = wid * CHUNK

    pltpu.sync_copy(lab_h.at[pl.ds(base, CHUNK)], lab_v)
    pltpu.sync_copy(nb_h.at[pl.ds(base, CHUNK)], nb_v)
    pltpu.sync_copy(targ_h.at[pl.ds(base, CHUNK)], targ_v)
    pltpu.sync_copy(cn_h, cn_v)
    pltpu.sync_copy(usn_h, usn_v)
    pltpu.sync_copy(neu_h, neu_v)

    lanes = lax.broadcasted_iota(jnp.int32, (16,), 0)
    zeros16 = jnp.zeros((16,), jnp.int32)
    ones16 = jnp.ones((16,), jnp.int32)

    hrows = (u * nbb) // 128

    def _zh(j, _):
        r = j // 8
        col = (j % 8) * 16
        hist_v[r, pl.ds(col, 16)] = zeros16
        return 0
    lax.fori_loop(0, hrows * 8, _zh, 0)

    @pl.when(sid == 0)
    def _():
        pltpu.sync_copy(hist_v, shared_v)
    plsc.subcore_barrier()

    for h in range(CHUNK // HALF):
        hbase = h * HALF

        def _p1(g, _):
            off = hbase + g * 16
            lab = lab_v[pl.ds(off, 16)]
            nb = nb_v[pl.ds(off, 16)]
            tg = targ_v[pl.ds(off, 16)]
            spike = base + off + lanes
            sp = jnp.minimum(spike, n - 1)
            valid = spike < n
            cands = []
            for c in range(C):
                cands.append(plsc.load_gather(cn_v, [lab * C + c]))
            for c in range(C):
                for s in range(NS):
                    cands.append(plsc.load_gather(usn_v, [cands[c] * NS + s]))
            cands.append(plsc.load_gather(neu_v, [nb * xp + tg]))
            for t in range(T):
                bin_ = cands[t] * nbb + nb
                plsc.addupdate_scatter(
                    hist_v, [jnp.right_shift(bin_, 7),
                             jnp.bitwise_and(bin_, 127)],
                    ones16, mask=valid)
                idx_v[pl.ds(t * HALF + g * 16, 16)] = (
                    jnp.right_shift(cands[t], 7) * (n * 128) + sp * 128
                    + jnp.bitwise_and(cands[t], 127))
                cand_v[pl.ds(t * HALF + g * 16, 16)] = cands[t]
            return 0
        lax.fori_loop(0, NGRP, _p1, 0)

        def _dma(j):
            src = score_h.at[idx_v.at[pl.ds(j * 128, 128)]]
            dst = gath_v.at[pl.ds(j * 128, 128)]
            return pltpu.make_async_copy(src, dst, sem)

        def _pg(j, _):
            _dma(j).start()

            @pl.when(j >= PIPE)
            def _():
                _dma(j - PIPE).wait()
            return 0
        lax.fori_loop(0, NDMA, _pg, 0)
        for k in range(PIPE):
            _dma(NDMA - PIPE + k).wait()

        def _p2(g, _):
            off16 = g * 16
            goff = hbase + off16
            gv = [gath_v[pl.ds(t * HALF + off16, 16)] for t in range(T)]
            ci = [cand_v[pl.ds(t * HALF + off16, 16)] for t in range(T)]
            cur = list(gv)
            neg = jnp.float32(-3e38)
            vals, ids = [], []
            for _k in range(C):
                m = cur[0]
                for t in range(1, T):
                    m = jnp.maximum(m, cur[t])
                ti = jnp.full((16,), T, jnp.int32)
                for t in range(T - 1, -1, -1):
                    ti = jnp.where(cur[t] == m, t, ti)
                idk = zeros16
                for t in range(T):
                    idk = jnp.where(ti == t, ci[t], idk)
                    cur[t] = jnp.where(ti == t, neg, cur[t])
                vals.append(m)
                ids.append(idk)
            e2 = jnp.exp(vals[1] - vals[0])
            e3 = jnp.exp(vals[2] - vals[0])
            s = 1.0 + e2 + e3
            q0_v[pl.ds(goff, 16)] = 1.0 / s
            q1_v[pl.ds(goff, 16)] = e2 / s
            q2_v[pl.ds(goff, 16)] = e3 / s
            i0_v[pl.ds(goff, 16)] = ids[0]
            i1_v[pl.ds(goff, 16)] = ids[1]
            i2_v[pl.ds(goff, 16)] = ids[2]
            return 0
        lax.fori_loop(0, NGRP, _p2, 0)

    pltpu.sync_copy(q0_v, q0_h.at[pl.ds(base, CHUNK)])
    pltpu.sync_copy(q1_v, q1_h.at[pl.ds(base, CHUNK)])
    pltpu.sync_copy(q2_v, q2_h.at[pl.ds(base, CHUNK)])
    pltpu.sync_copy(i0_v, i0_h.at[pl.ds(base, CHUNK)])
    pltpu.sync_copy(i1_v, i1_h.at[pl.ds(base, CHUNK)])
    pltpu.sync_copy(i2_v, i2_h.at[pl.ds(base, CHUNK)])

    for j in range(hrows // 128):
        for k in range(8):
            rowidx_v[j, pl.ds(k * 16, 16)] = j * 128 + k * 16 + lanes
    for j in range(hrows // 128):
        pltpu.sync_copy(hist_v.at[pl.ds(j * 128, 128)],
                        shared_v.at[rowidx_v.at[j]], add=True)
    plsc.subcore_barrier()

    @pl.when(sid == 0)
    def _():
        pltpu.sync_copy(shared_v, cnt_h.at[cid])


@jax.jit
def _run(x, means, log_proportions, labels, neighborhood_ids,
         closest_neighbors, unit_search_neighbors, explore_targs,
         neighborhood_explore_units):
    n, d = x.shape
    u = means.shape[0]
    nbb, xp = neighborhood_explore_units.shape
    npad = NW * CHUNK
    hrows = (u * nbb) // 128

    mt = means.T
    lp = log_proportions.reshape(1, u)
    score = _make_score(n, d, u, 2000)(x, mt, lp)

    pad = (0, npad - n)
    lab_p = jnp.pad(labels, pad)
    nb_p = jnp.pad(neighborhood_ids, pad)
    targ_p = jnp.pad(explore_targs.reshape(-1), pad)

    sc = functools.partial(
        pl.kernel,
        mesh=plsc.VectorSubcoreMesh(core_axis_name="c", subcore_axis_name="s"),
        compiler_params=pltpu.CompilerParams(needs_layout_passes=False),
        out_type=[
            jax.ShapeDtypeStruct((npad,), jnp.float32),
            jax.ShapeDtypeStruct((npad,), jnp.float32),
            jax.ShapeDtypeStruct((npad,), jnp.float32),
            jax.ShapeDtypeStruct((npad,), jnp.int32),
            jax.ShapeDtypeStruct((npad,), jnp.int32),
            jax.ShapeDtypeStruct((npad,), jnp.int32),
            jax.ShapeDtypeStruct((2, hrows, 128), jnp.int32),
        ],
        scratch_types=[
            pltpu.VMEM((CHUNK,), jnp.int32),
            pltpu.VMEM((CHUNK,), jnp.int32),
            pltpu.VMEM((CHUNK,), jnp.int32),
            pltpu.VMEM((u * C,), jnp.int32),
            pltpu.VMEM((u * NS,), jnp.int32),
            pltpu.VMEM((nbb * xp,), jnp.int32),
            pltpu.VMEM((HALF * T,), jnp.int32),
            pltpu.VMEM((HALF * T,), jnp.int32),
            pltpu.VMEM((HALF * T,), jnp.float32),
            pltpu.VMEM((CHUNK,), jnp.float32),
            pltpu.VMEM((CHUNK,), jnp.float32),
            pltpu.VMEM((CHUNK,), jnp.float32),
            pltpu.VMEM((CHUNK,), jnp.int32),
            pltpu.VMEM((CHUNK,), jnp.int32),
            pltpu.VMEM((CHUNK,), jnp.int32),
            pltpu.VMEM((hrows, 128), jnp.int32),
            pltpu.VMEM((hrows // 128, 128), jnp.int32),
            pltpu.VMEM_SHARED((hrows, 128), jnp.int32),
            pltpu.SemaphoreType.DMA,
        ],
    )(functools.partial(_sc_body, n, u, nbb, xp))

    q0, q1, q2, i0, i1, i2, cnt = sc(
        lab_p, nb_p, targ_p,
        closest_neighbors.reshape(-1), unit_search_neighbors.reshape(-1),
        neighborhood_explore_units.reshape(-1), score.reshape(-1))

    q = jnp.stack([q0[:n], q1[:n], q2[:n]], axis=1)
    top = jnp.stack([i0[:n], i1[:n], i2[:n]], axis=1)
    counts = cnt.sum(axis=0).reshape(u, nbb)
    return q, top, counts


def kernel(x, means, log_proportions, labels, neighborhood_ids,
           closest_neighbors, unit_search_neighbors, explore_targs,
           neighborhood_explore_units):
    return _run(x, means, log_proportions, labels, neighborhood_ids,
                closest_neighbors, unit_search_neighbors, explore_targs,
                neighborhood_explore_units)


# speedup vs baseline: 69.5144x; 1.1864x over previous
import functools

import jax
import jax.numpy as jnp
from jax import lax
from jax.experimental import pallas as pl
from jax.experimental.pallas import tpu as pltpu
from jax.experimental.pallas import tpu_sc as plsc

C = 3
NS = 2
T = C + C * NS + 1

NW = 32
CHUNK = 3200
QC = 640
NST = CHUNK // QC
QGRP = QC // 16
QDMA = QC * T // 128


def _score_body(x_ref, mt_ref, lp_ref, out_ref):
    mt = mt_ref[:]
    mu2 = jnp.sum(mt * mt, axis=0, keepdims=True)
    score = jnp.dot(x_ref[:], mt, preferred_element_type=jnp.float32) \
        + (lp_ref[:] - 0.5 * mu2)
    for q in range(out_ref.shape[0]):
        out_ref[q, :, :] = score[:, q * 128:(q + 1) * 128]


def _make_score(n, d, u, bn):
    nq = u // 128
    return pl.pallas_call(
        _score_body,
        grid=(n // bn,),
        in_specs=[
            pl.BlockSpec((bn, d), lambda i: (i, 0)),
            pl.BlockSpec((d, u), lambda i: (0, 0)),
            pl.BlockSpec((1, u), lambda i: (0, 0)),
        ],
        out_specs=pl.BlockSpec((nq, bn, 128), lambda i: (0, i, 0)),
        out_shape=jax.ShapeDtypeStruct((nq, n, 128), jnp.float32),
        compiler_params=pltpu.CompilerParams(
            dimension_semantics=("parallel",)),
    )


def _sc_body(n, u, nbb, xp,
             lab_h, nb_h, targ_h, cn_h, usn_h, neu_h, score_h,
             q0_h, q1_h, q2_h, i0_h, i1_h, i2_h, cnt_h,
             lab_v, nb_v, targ_v, cn_v, usn_v, neu_v,
             idx_v, idx2_v, cand_v, cand2_v, gath_v, gath2_v,
             q0_v, q1_v, q2_v, i0_v, i1_v, i2_v,
             hist_v, rowidx_v, shared_v, sem, sem2):
    sid = lax.axis_index("s")
    cid = lax.axis_index("c")
    wid = sid * 2 + cid
    base = wid * CHUNK

    pltpu.sync_copy(lab_h.at[pl.ds(base, CHUNK)], lab_v)
    pltpu.sync_copy(nb_h.at[pl.ds(base, CHUNK)], nb_v)
    pltpu.sync_copy(targ_h.at[pl.ds(base, CHUNK)], targ_v)
    pltpu.sync_copy(cn_h, cn_v)
    pltpu.sync_copy(usn_h, usn_v)
    pltpu.sync_copy(neu_h, neu_v)

    lanes = lax.broadcasted_iota(jnp.int32, (16,), 0)
    zeros16 = jnp.zeros((16,), jnp.int32)
    ones16 = jnp.ones((16,), jnp.int32)

    hrows = (u * nbb) // 128

    def _zh(j, _):
        r = j // 8
        col = (j % 8) * 16
        hist_v[r, pl.ds(col, 16)] = zeros16
        return 0
    lax.fori_loop(0, hrows * 8, _zh, 0)

    @pl.when(sid == 0)
    def _():
        pltpu.sync_copy(hist_v, shared_v)
    plsc.subcore_barrier()

    idx_b = [idx_v, idx2_v]
    cand_b = [cand_v, cand2_v]
    gath_b = [gath_v, gath2_v]
    sem_b = [sem, sem2]

    def _p1(q, par):
        def body(g, _):
            off = q * QC + g * 16
            lab = lab_v[pl.ds(off, 16)]
            nb = nb_v[pl.ds(off, 16)]
            tg = targ_v[pl.ds(off, 16)]
            spike = base + off + lanes
            sp = jnp.minimum(spike, n - 1)
            valid = spike < n
            cands = []
            for c in range(C):
                cands.append(plsc.load_gather(cn_v, [lab * C + c]))
            for c in range(C):
                for s in range(NS):
                    cands.append(plsc.load_gather(usn_v, [cands[c] * NS + s]))
            cands.append(plsc.load_gather(neu_v, [nb * xp + tg]))
            for t in range(T):
                bin_ = cands[t] * nbb + nb
                plsc.addupdate_scatter(
                    hist_v, [jnp.right_shift(bin_, 7),
                             jnp.bitwise_and(bin_, 127)],
                    ones16, mask=valid)
                idx_b[par][pl.ds(t * QC + g * 16, 16)] = (
                    jnp.right_shift(cands[t], 7) * (n * 128) + sp * 128
                    + jnp.bitwise_and(cands[t], 127))
                cand_b[par][pl.ds(t * QC + g * 16, 16)] = cands[t]
            return 0
        lax.fori_loop(0, QGRP, body, 0)

    def _dma(j, par):
        src = score_h.at[idx_b[par].at[pl.ds(j * 128, 128)]]
        dst = gath_b[par].at[pl.ds(j * 128, 128)]
        return pltpu.make_async_copy(src, dst, sem_b[par])

    def _fire(par):
        def body(j, _):
            _dma(j, par).start()
            return 0
        lax.fori_loop(0, QDMA, body, 0)

    def _drain(par):
        def body(j, _):
            _dma(j, par).wait()
            return 0
        lax.fori_loop(0, QDMA, body, 0)

    def _p2(q, par):
        def body(g, _):
            off16 = g * 16
            goff = q * QC + off16
            cur = [gath_b[par][pl.ds(t * QC + off16, 16)] for t in range(T)]
            neg = jnp.float32(-3e38)
            vals, ids = [], []
            for _k in range(C):
                m = cur[0]
                for t in range(1, T):
                    m = jnp.maximum(m, cur[t])
                ti = jnp.full((16,), T, jnp.int32)
                for t in range(T - 1, -1, -1):
                    ti = jnp.where(cur[t] == m, t, ti)
                ids.append(plsc.load_gather(cand_b[par],
                                            [ti * QC + off16 + lanes]))
                for t in range(T):
                    cur[t] = jnp.where(ti == t, neg, cur[t])
                vals.append(m)
            e2 = jnp.exp(vals[1] - vals[0])
            e3 = jnp.exp(vals[2] - vals[0])
            s = 1.0 + e2 + e3
            q0_v[pl.ds(goff, 16)] = 1.0 / s
            q1_v[pl.ds(goff, 16)] = e2 / s
            q2_v[pl.ds(goff, 16)] = e3 / s
            i0_v[pl.ds(goff, 16)] = ids[0]
            i1_v[pl.ds(goff, 16)] = ids[1]
            i2_v[pl.ds(goff, 16)] = ids[2]
            return 0
        lax.fori_loop(0, QGRP, body, 0)

    for q in range(NST):
        par = q % 2
        _p1(q, par)
        _fire(par)
        if q > 0:
            _drain(1 - par)
            _p2(q - 1, 1 - par)
    _drain((NST - 1) % 2)
    _p2(NST - 1, (NST - 1) % 2)

    pltpu.sync_copy(q0_v, q0_h.at[pl.ds(base, CHUNK)])
    pltpu.sync_copy(q1_v, q1_h.at[pl.ds(base, CHUNK)])
    pltpu.sync_copy(q2_v, q2_h.at[pl.ds(base, CHUNK)])
    pltpu.sync_copy(i0_v, i0_h.at[pl.ds(base, CHUNK)])
    pltpu.sync_copy(i1_v, i1_h.at[pl.ds(base, CHUNK)])
    pltpu.sync_copy(i2_v, i2_h.at[pl.ds(base, CHUNK)])

    for j in range(hrows // 128):
        for k in range(8):
            rowidx_v[j, pl.ds(k * 16, 16)] = j * 128 + k * 16 + lanes
    for j in range(hrows // 128):
        pltpu.sync_copy(hist_v.at[pl.ds(j * 128, 128)],
                        shared_v.at[rowidx_v.at[j]], add=True)
    plsc.subcore_barrier()

    @pl.when(sid == 0)
    def _():
        pltpu.sync_copy(shared_v, cnt_h.at[cid])


@jax.jit
def _run(x, means, log_proportions, labels, neighborhood_ids,
         closest_neighbors, unit_search_neighbors, explore_targs,
         neighborhood_explore_units):
    n, d = x.shape
    u = means.shape[0]
    nbb, xp = neighborhood_explore_units.shape
    npad = NW * CHUNK
    hrows = (u * nbb) // 128

    mt = means.T
    lp = log_proportions.reshape(1, u)
    score = _make_score(n, d, u, 2000)(x, mt, lp)

    pad = (0, npad - n)
    lab_p = jnp.pad(labels, pad)
    nb_p = jnp.pad(neighborhood_ids, pad)
    targ_p = jnp.pad(explore_targs.reshape(-1), pad)

    sc = functools.partial(
        pl.kernel,
        mesh=plsc.VectorSubcoreMesh(core_axis_name="c", subcore_axis_name="s"),
        compiler_params=pltpu.CompilerParams(needs_layout_passes=False),
        out_type=[
            jax.ShapeDtypeStruct((npad,), jnp.float32),
            jax.ShapeDtypeStruct((npad,), jnp.float32),
            jax.ShapeDtypeStruct((npad,), jnp.float32),
            jax.ShapeDtypeStruct((npad,), jnp.int32),
            jax.ShapeDtypeStruct((npad,), jnp.int32),
            jax.ShapeDtypeStruct((npad,), jnp.int32),
            jax.ShapeDtypeStruct((2, hrows, 128), jnp.int32),
        ],
        scratch_types=[
            pltpu.VMEM((CHUNK,), jnp.int32),
            pltpu.VMEM((CHUNK,), jnp.int32),
            pltpu.VMEM((CHUNK,), jnp.int32),
            pltpu.VMEM((u * C,), jnp.int32),
            pltpu.VMEM((u * NS,), jnp.int32),
            pltpu.VMEM((nbb * xp,), jnp.int32),
            pltpu.VMEM((QC * T,), jnp.int32),
            pltpu.VMEM((QC * T,), jnp.int32),
            pltpu.VMEM((QC * T,), jnp.int32),
            pltpu.VMEM((QC * T,), jnp.int32),
            pltpu.VMEM((QC * T,), jnp.float32),
            pltpu.VMEM((QC * T,), jnp.float32),
            pltpu.VMEM((CHUNK,), jnp.float32),
            pltpu.VMEM((CHUNK,), jnp.float32),
            pltpu.VMEM((CHUNK,), jnp.float32),
            pltpu.VMEM((CHUNK,), jnp.int32),
            pltpu.VMEM((CHUNK,), jnp.int32),
            pltpu.VMEM((CHUNK,), jnp.int32),
            pltpu.VMEM((hrows, 128), jnp.int32),
            pltpu.VMEM((hrows // 128, 128), jnp.int32),
            pltpu.VMEM_SHARED((hrows, 128), jnp.int32),
            pltpu.SemaphoreType.DMA,
            pltpu.SemaphoreType.DMA,
        ],
    )(functools.partial(_sc_body, n, u, nbb, xp))

    q0, q1, q2, i0, i1, i2, cnt = sc(
        lab_p, nb_p, targ_p,
        closest_neighbors.reshape(-1), unit_search_neighbors.reshape(-1),
        neighborhood_explore_units.reshape(-1), score.reshape(-1))

    q = jnp.stack([q0[:n], q1[:n], q2[:n]], axis=1)
    top = jnp.stack([i0[:n], i1[:n], i2[:n]], axis=1)
    counts = cnt.sum(axis=0).reshape(u, nbb)
    return q, top, counts


def kernel(x, means, log_proportions, labels, neighborhood_ids,
           closest_neighbors, unit_search_neighbors, explore_targs,
           neighborhood_explore_units):
    return _run(x, means, log_proportions, labels, neighborhood_ids,
                closest_neighbors, unit_search_neighbors, explore_targs,
                neighborhood_explore_units)
